# baseline (speedup 1.0000x reference)
import jax
import jax.numpy as jnp
from jax.experimental import pallas as pl

N = 10000
E = 160000
D = 128
M = 200
HEADS = 2
OUT = 128
H = 256
SLOPE = 0.2


def _mm(a, w, b=None, block=1000):
    rows, k = a.shape
    n = w.shape[1]
    block = min(block, rows)

    def kern(a_ref, w_ref, o_ref):
        o_ref[...] = jnp.dot(a_ref[...], w_ref[...],
                             preferred_element_type=jnp.float32)

    out = pl.pallas_call(
        kern,
        grid=(pl.cdiv(rows, block),),
        in_specs=[pl.BlockSpec((block, k), lambda i: (i, 0)),
                  pl.BlockSpec((k, n), lambda i: (0, 0))],
        out_specs=pl.BlockSpec((block, n), lambda i: (i, 0)),
        out_shape=jax.ShapeDtypeStruct((rows, n), jnp.float32),
    )(a, w)
    if b is not None:
        out = out + b
    return out


def _normalize(x, axis):
    nrm = jnp.linalg.norm(x, ord=2, axis=axis, keepdims=True)
    return x / jnp.maximum(nrm, 1e-12)


def _rel_att(h_ijk, ends, n, W, att, heads, out):
    c = _mm(h_ijk, W).reshape(-1, heads, out)
    a = jnp.sum(att * c, axis=2)[:, :, None]
    b = -jax.nn.leaky_relu(a, SLOPE)
    e = jnp.exp(b)
    rs = jax.ops.segment_sum(e, ends, num_segments=n)
    alpha = e / rs[ends]
    h = jax.ops.segment_sum(alpha * c, ends, num_segments=n)
    h = jax.nn.elu(h)
    h = _normalize(h, 2)
    return h.reshape(n, heads * out)


def _merge(hi, ho, Wi, bi, Wo, bo, Wl, bl):
    hi = _mm(hi, Wi, bi)
    ho = _mm(ho, Wo, bo)
    lam = jax.nn.sigmoid(jnp.concatenate([hi, ho], axis=1) @ Wl + bl)
    h = lam * hi + (1.0 - lam) * ho
    return _normalize(jax.nn.elu(h), 1)


def _relation(h_ijk, g, edge_type, Wr, br, Wf, bf):
    ge = jax.ops.segment_sum(h_ijk, edge_type, num_segments=M)
    ge = jax.nn.elu(_normalize(ge, 1))
    return _mm(g, Wr, br) + _mm(ge, Wf, bf)


def kernel(x, g, edge_index, edge_type, W_in1, att_in1, W_out1, att_out1,
           Wm1_i, bm1_i, Wm1_o, bm1_o, Wm1_l, bm1_l, W_rel, b_rel, W_rf,
           b_rf, W_in2, att_in2, W_out2, att_out2, Wm2_i, bm2_i, Wm2_o,
           bm2_o, Wm2_l, bm2_l, W_ent):
    row = edge_index[0]
    col = edge_index[1]
    h_ijk = jnp.concatenate([x[row], x[col], g[edge_type]], axis=1)
    h_in = _rel_att(h_ijk, col, N, W_in1, att_in1, HEADS, OUT)
    h_out = _rel_att(h_ijk, row, N, W_out1, att_out1, HEADS, OUT)
    h1 = _merge(h_in, h_out, Wm1_i, bm1_i, Wm1_o, bm1_o, Wm1_l, bm1_l)
    g1 = _relation(h_ijk, g, edge_type, W_rel, b_rel, W_rf, b_rf)
    h_ijk2 = jnp.concatenate([h1[row], h1[col], g1[edge_type]], axis=1)
    h_in2 = _rel_att(h_ijk2, col, N, W_in2, att_in2, 1, H)
    h_out2 = _rel_att(h_ijk2, row, N, W_out2, att_out2, 1, H)
    h2 = _merge(h_in2, h_out2, Wm2_i, bm2_i, Wm2_o, bm2_o, Wm2_l, bm2_l)
    h_final = _mm(x, W_ent) + h2
    return _normalize(h_final, 1)
